# baseline (speedup 1.0000x reference)
import jax
import jax.numpy as jnp
from jax import lax
from jax.experimental import pallas as pl
from jax.experimental.pallas import tpu as pltpu
from jax.experimental.pallas import tpu_sc as plsc

_H = 32
_T = 2209
_R = 576
_C = 576
_NC, _NS, _L = 2, 16, 16
_HG = 8
_WPG = 8
_SR = 8
_SE = _SR * _C
_NST = _R // _SR
_SPW = _NST // _WPG
_CPR = _C // _L


def _body(tab_hbm, idx_hbm, out_hbm,
          tab_v, ix0, ix1, ob0, ob1, semi0, semi1, semo0, semo1):
    w = lax.axis_index("s") * _NC + lax.axis_index("c")
    hg = w // _WPG
    wk = w % _WPG
    hbase = pl.multiple_of(hg * _HG, _HG)
    pltpu.sync_copy(tab_hbm.at[pl.ds(hbase, _HG), :], tab_v)

    def _idx_src(j):
        off = pl.multiple_of((wk + j * _WPG) * _SE, _SE)
        return idx_hbm.at[pl.ds(off, _SE)]

    def _dst(j):
        r0 = pl.multiple_of((wk + j * _WPG) * _SR, _SR)
        return out_hbm.at[pl.ds(hbase, _HG), pl.ds(r0, _SR), :]

    def _fill(ix, ob):
        for rr in range(_SR):
            @plsc.parallel_loop(0, _CPR, 1, unroll=3)
            def _chunk(c):
                iv = ix[pl.ds(rr * _C + c * _L, _L)]
                s = pl.ds(c * _L, _L)
                for hl in range(_HG):
                    hv = jnp.full((_L,), hl, jnp.int32)
                    ob[hl, rr, s] = plsc.load_gather(tab_v, [hv, iv])

    pltpu.async_copy(_idx_src(0), ix0, semi0)

    def _stripe(j, ix, semi, ob, semo, nxt_ix, nxt_semi, first):
        pltpu.make_async_copy(_idx_src(j), ix, semi).wait()
        pltpu.async_copy(_idx_src(j + 1), nxt_ix, nxt_semi)

        @pl.when(jnp.logical_not(first))
        def _():
            pltpu.make_async_copy(ob, _dst(j), semo).wait()

        _fill(ix, ob)
        pltpu.async_copy(ob, _dst(j), semo)

    def _pair(jp, carry):
        _stripe(2 * jp, ix0, semi0, ob0, semo0, ix1, semi1, jp == 0)
        _stripe(2 * jp + 1, ix1, semi1, ob1, semo1, ix0, semi0, jp == 0)
        return carry

    lax.fori_loop(0, (_SPW - 1) // 2, _pair, 0)

    jt = _SPW - 1
    pltpu.make_async_copy(_idx_src(jt), ix0, semi0).wait()
    pltpu.make_async_copy(ob0, _dst(jt), semo0).wait()
    _fill(ix0, ob0)
    pltpu.async_copy(ob0, _dst(jt), semo0)
    pltpu.make_async_copy(ob0, _dst(jt), semo0).wait()
    pltpu.make_async_copy(ob1, _dst(jt), semo1).wait()


def kernel(relative_position_bias_table, relative_index):
    tab_t = relative_position_bias_table.T
    idx_flat = relative_index.reshape(-1)
    mesh = plsc.VectorSubcoreMesh(core_axis_name="c", subcore_axis_name="s")
    return pl.kernel(
        _body,
        out_type=jax.ShapeDtypeStruct((_H, _R, _C), jnp.float32),
        mesh=mesh,
        scratch_types=[
            pltpu.VMEM((_HG, _T), jnp.float32),
            pltpu.VMEM((_SE,), jnp.int32),
            pltpu.VMEM((_SE,), jnp.int32),
            pltpu.VMEM((_HG, _SR, _C), jnp.float32),
            pltpu.VMEM((_HG, _SR, _C), jnp.float32),
            pltpu.SemaphoreType.DMA,
            pltpu.SemaphoreType.DMA,
            pltpu.SemaphoreType.DMA,
            pltpu.SemaphoreType.DMA,
        ],
        compiler_params=pltpu.CompilerParams(needs_layout_passes=False),
    )(tab_t, idx_flat)


# speedup vs baseline: 1.0143x; 1.0143x over previous
import jax
import jax.numpy as jnp
from jax import lax
from jax.experimental import pallas as pl
from jax.experimental.pallas import tpu as pltpu
from jax.experimental.pallas import tpu_sc as plsc

_H = 32
_T = 2209
_R = 576
_C = 576
_NC, _NS, _L = 2, 16, 16
_HG = 8
_WPG = 8
_SR = 8
_SE = _SR * _C
_NST = _R // _SR
_SPW = _NST // _WPG
_CPR = _C // _L


def _body(tab_hbm, idx_hbm, out_hbm,
          tab_v, ix0, ix1, ob0, ob1, semi0, semi1, semo0, semo1):
    w = lax.axis_index("s") * _NC + lax.axis_index("c")
    hg = w // _WPG
    wk = w % _WPG
    hbase = pl.multiple_of(hg * _HG, _HG)

    def _idx_src(j):
        off = pl.multiple_of((wk + j * _WPG) * _SE, _SE)
        return idx_hbm.at[pl.ds(off, _SE)]

    def _dst(j):
        r0 = pl.multiple_of((wk + j * _WPG) * _SR, _SR)
        return out_hbm.at[pl.ds(hbase, _HG), pl.ds(r0, _SR), :]

    def _fill(ix, ob):
        for rr in range(_SR):
            @plsc.parallel_loop(0, _CPR, 1, unroll=2)
            def _chunk(c):
                iv = ix[pl.ds(rr * _C + c * _L, _L)]
                s = pl.ds(c * _L, _L)
                for hl in range(_HG):
                    hv = jnp.full((_L,), hl, jnp.int32)
                    ob[hl, rr, s] = plsc.load_gather(tab_v, [hv, iv])

    pltpu.async_copy(_idx_src(0), ix0, semi0)
    pltpu.sync_copy(tab_hbm.at[pl.ds(hbase, _HG), :], tab_v)

    def _stripe(j, ix, semi, ob, semo, nxt_ix, nxt_semi, first):
        pltpu.make_async_copy(_idx_src(j), ix, semi).wait()
        pltpu.async_copy(_idx_src(j + 1), nxt_ix, nxt_semi)

        @pl.when(jnp.logical_not(first))
        def _():
            pltpu.make_async_copy(ob, _dst(j), semo).wait()

        _fill(ix, ob)
        pltpu.async_copy(ob, _dst(j), semo)

    def _pair(jp, carry):
        _stripe(2 * jp, ix0, semi0, ob0, semo0, ix1, semi1, jp == 0)
        _stripe(2 * jp + 1, ix1, semi1, ob1, semo1, ix0, semi0, jp == 0)
        return carry

    lax.fori_loop(0, (_SPW - 1) // 2, _pair, 0)

    jt = _SPW - 1
    pltpu.make_async_copy(_idx_src(jt), ix0, semi0).wait()
    pltpu.make_async_copy(ob0, _dst(jt), semo0).wait()
    _fill(ix0, ob0)
    pltpu.async_copy(ob0, _dst(jt), semo0)
    pltpu.make_async_copy(ob0, _dst(jt), semo0).wait()
    pltpu.make_async_copy(ob1, _dst(jt), semo1).wait()


def kernel(relative_position_bias_table, relative_index):
    tab_t = relative_position_bias_table.T
    idx_flat = relative_index.reshape(-1)
    mesh = plsc.VectorSubcoreMesh(core_axis_name="c", subcore_axis_name="s")
    return pl.kernel(
        _body,
        out_type=jax.ShapeDtypeStruct((_H, _R, _C), jnp.float32),
        mesh=mesh,
        scratch_types=[
            pltpu.VMEM((_HG, _T), jnp.float32),
            pltpu.VMEM((_SE,), jnp.int32),
            pltpu.VMEM((_SE,), jnp.int32),
            pltpu.VMEM((_HG, _SR, _C), jnp.float32),
            pltpu.VMEM((_HG, _SR, _C), jnp.float32),
            pltpu.SemaphoreType.DMA,
            pltpu.SemaphoreType.DMA,
            pltpu.SemaphoreType.DMA,
            pltpu.SemaphoreType.DMA,
        ],
        compiler_params=pltpu.CompilerParams(needs_layout_passes=False),
    )(tab_t, idx_flat)
